# Initial kernel scaffold; baseline (speedup 1.0000x reference)
import jax
import jax.numpy as jnp
from jax.experimental import pallas as pl


def kernel(x_user, x_item, ei_user_to_item, ei_item_to_user, W0_user, b0_user, W0_item, b0_item, W1_user, b1_user, W1_item, b1_item, W2_user, b2_user, W2_item, b2_item):
    raise NotImplementedError("write your pallas kernel here")


# speedup vs baseline: 2.0147x; 2.0147x over previous
import jax
import jax.numpy as jnp
from jax import lax
from jax.experimental import pallas as pl
from jax.experimental.pallas import tpu as pltpu
from jax.experimental.pallas import tpu_sc as plsc

_N = 50000
_E = 500000
_D = 128
_NQ = 8
_DQ = _D // _NQ
_SUB = 125
_NSUB = 8
_CH = _SUB * _NSUB
_NCH = _E // _CH
_NS = 16
_NPAD = 50048
_RPT = _NPAD // _NS
_ITERS = (_NCH + _NS - 1) // _NS


def _seg_side(s, tbls, src_r, dst_r, outs, zeros_r, idx_s, idx_d, rows, acc,
              sem_g, sem_s):
  base = s * _RPT
  pltpu.sync_copy(zeros_r, acc.at[pl.ds(base, _RPT)])
  for q in range(_NQ):
    plsc.subcore_barrier()

    def chunk(i, carry):
      g = i * _NS + s

      @pl.when(g < _NCH)
      def _():
        pltpu.sync_copy(src_r.at[g], idx_s)
        pltpu.sync_copy(dst_r.at[g], idx_d)
        gd = [pltpu.make_async_copy(tbls[q].at[idx_s.at[j]],
                                    rows.at[pl.ds(j * _SUB, _SUB)], sem_g)
              for j in range(_NSUB)]
        for d in gd:
          d.start()
        for d in gd:
          d.wait()
        sd = [pltpu.make_async_copy(rows.at[pl.ds(j * _SUB, _SUB)],
                                    acc.at[idx_d.at[j]], sem_s)
              for j in range(_NSUB)]
        for d in sd:
          d.start(add=True)
        for d in sd:
          d.wait()

      return carry

    lax.fori_loop(0, _ITERS, chunk, 0)
    plsc.subcore_barrier()
    pltpu.sync_copy(acc.at[pl.ds(base, _RPT)], outs[q].at[pl.ds(base, _RPT)])
    if q < _NQ - 1:
      pltpu.sync_copy(zeros_r, acc.at[pl.ds(base, _RPT)])


def _sc_body(*refs):
  tu = refs[0:_NQ]
  ti = refs[_NQ:2 * _NQ]
  src_u, dst_u, src_i, dst_i, zeros_r = refs[2 * _NQ:2 * _NQ + 5]
  ou = refs[2 * _NQ + 5:3 * _NQ + 5]
  oi = refs[3 * _NQ + 5:4 * _NQ + 5]
  idx_s, idx_d, rows, acc, sem_g, sem_s = refs[4 * _NQ + 5:]
  c = lax.axis_index("c")
  s = lax.axis_index("s")

  @pl.when(c == 0)
  def _():
    _seg_side(s, tu, src_u, dst_u, ou, zeros_r, idx_s, idx_d, rows, acc,
              sem_g, sem_s)

  @pl.when(c == 1)
  def _():
    _seg_side(s, ti, src_i, dst_i, oi, zeros_r, idx_s, idx_d, rows, acc,
              sem_g, sem_s)


def _make_sc_call():
  f32 = jnp.float32
  return pl.kernel(
      _sc_body,
      out_type=[jax.ShapeDtypeStruct((_NPAD, _DQ), f32)] * (2 * _NQ),
      mesh=plsc.VectorSubcoreMesh(core_axis_name="c", subcore_axis_name="s"),
      scratch_types=[
          pltpu.VMEM((_NSUB, _SUB), jnp.int32),
          pltpu.VMEM((_NSUB, _SUB), jnp.int32),
          pltpu.VMEM((_CH, _DQ), f32),
          pltpu.VMEM_SHARED((_NPAD, _DQ), f32),
          pltpu.SemaphoreType.DMA,
          pltpu.SemaphoreType.DMA,
      ],
      compiler_params=pltpu.CompilerParams(use_tc_tiling_on_sc=False),
  )


_R = 1000


def _taps_body(*refs):
  xu, xi = refs[0], refs[1]
  mu1 = refs[2:2 + _NQ]
  mu2 = refs[2 + _NQ:2 + 2 * _NQ]
  mi1 = refs[2 + 2 * _NQ:2 + 3 * _NQ]
  mi2 = refs[2 + 3 * _NQ:2 + 4 * _NQ]
  w0u, w1u, w2u, w0i, w1i, w2i, bu, bi = refs[2 + 4 * _NQ:2 + 4 * _NQ + 8]
  zu, zi = refs[-2], refs[-1]

  def side(x, m1, m2, w0, w1, w2, b, z):
    acc = jnp.dot(x[...], w0[...], preferred_element_type=jnp.float32)
    x1 = jnp.concatenate([m[...] for m in m1], axis=1)
    acc = acc + jnp.dot(x1, w1[...], preferred_element_type=jnp.float32)
    x2 = jnp.concatenate([m[...] for m in m2], axis=1)
    acc = acc + jnp.dot(x2, w2[...], preferred_element_type=jnp.float32)
    z[...] = acc + jnp.sum(b[...], axis=0, keepdims=True)

  side(xu, mu1, mu2, w0u, w1u, w2u, bu, zu)
  side(xi, mi1, mi2, w0i, w1i, w2i, bi, zi)


def _make_tc_call():
  f32 = jnp.float32
  blk = lambda shape: pl.BlockSpec(shape, lambda i: (i, 0))
  rep = lambda shape: pl.BlockSpec(shape, lambda i: (0, 0))
  in_specs = ([blk((_R, _D))] * 2 + [blk((_R, _DQ))] * (4 * _NQ) +
              [rep((_D, _D))] * 6 + [rep((3, _D))] * 2)
  return pl.pallas_call(
      _taps_body,
      grid=(_N // _R,),
      in_specs=in_specs,
      out_specs=[blk((_R, _D))] * 2,
      out_shape=[jax.ShapeDtypeStruct((_N, _D), f32)] * 2,
  )


def kernel(x_user, x_item, ei_user_to_item, ei_item_to_user,
           W0_user, b0_user, W0_item, b0_item,
           W1_user, b1_user, W1_item, b1_item,
           W2_user, b2_user, W2_item, b2_item):
  i32 = jnp.int32
  src_u = ei_item_to_user[0].astype(i32).reshape(_NCH, _NSUB, _SUB)
  dst_u = ei_item_to_user[1].astype(i32).reshape(_NCH, _NSUB, _SUB)
  src_i = ei_user_to_item[0].astype(i32).reshape(_NCH, _NSUB, _SUB)
  dst_i = ei_user_to_item[1].astype(i32).reshape(_NCH, _NSUB, _SUB)
  tu = [x_item[:, q * _DQ:(q + 1) * _DQ] for q in range(_NQ)]
  ti = [x_user[:, q * _DQ:(q + 1) * _DQ] for q in range(_NQ)]
  zeros = jnp.zeros((_RPT, _DQ), jnp.float32)

  sc = _make_sc_call()
  o1 = sc(*tu, *ti, src_u, dst_u, src_i, dst_i, zeros)
  mu1, mi1 = o1[:_NQ], o1[_NQ:]
  o2 = sc(*mi1, *mu1, src_u, dst_u, src_i, dst_i, zeros)
  mu2, mi2 = o2[:_NQ], o2[_NQ:]

  bu = jnp.stack([b0_user, b1_user, b2_user])
  bi = jnp.stack([b0_item, b1_item, b2_item])
  z_user, z_item = _make_tc_call()(
      x_user, x_item, *mu1, *mu2, *mi1, *mi2,
      W0_user, W1_user, W2_user, W0_item, W1_item, W2_item, bu, bi)
  return (z_user, z_item)
